# Initial kernel scaffold; baseline (speedup 1.0000x reference)
import jax
import jax.numpy as jnp
from jax.experimental import pallas as pl


def kernel(x, edge_index, W1, b1, W2, b2, Wl, bl):
    raise NotImplementedError("write your pallas kernel here")


# speedup vs baseline: 10.7772x; 10.7772x over previous
import functools

import jax
import jax.numpy as jnp
from jax import lax
from jax.experimental import pallas as pl
from jax.experimental.pallas import tpu as pltpu
from jax.experimental.pallas import tpu_sc as plsc

N = 10000
D = 128
H = 128
C = 10
E = 320000

NP = 10240
NW = 32
CH = 128
NCH = 80
EPAD = NW * NCH * CH
ROWS_PER_SUB = NP // 16

_mesh = plsc.VectorSubcoreMesh(core_axis_name="c", subcore_axis_name="s")


@functools.partial(
    pl.kernel,
    out_type=(
        jax.ShapeDtypeStruct((2, NP), jnp.float32),
        jax.ShapeDtypeStruct((2, NP), jnp.float32),
    ),
    mesh=_mesh,
    scratch_types=[
        pltpu.VMEM((NCH + 2, CH), jnp.int32),
        pltpu.VMEM((NCH + 2, CH), jnp.int32),
        pltpu.VMEM((CH,), jnp.float32),
        pltpu.VMEM((ROWS_PER_SUB,), jnp.float32),
        pltpu.VMEM_SHARED((NP,), jnp.float32),
        pltpu.VMEM_SHARED((NP,), jnp.float32),
        pltpu.SemaphoreType.DMA,
    ],
)
def _sc_deg(src_hbm, dst_hbm, os_hbm, od_hbm,
            idx_s, idx_d, ones_v, zv, dssp, ddsp, sem):
    c = lax.axis_index("c")
    s = lax.axis_index("s")
    wid = s * 2 + c
    pltpu.sync_copy(src_hbm.at[wid], idx_s)
    pltpu.sync_copy(dst_hbm.at[wid], idx_d)

    def fill_ones(r, _):
        ones_v[pl.ds(r * 16, 16)] = jnp.ones((16,), jnp.float32)
        return 0

    lax.fori_loop(0, CH // 16, fill_ones, 0)

    def fill_zero(r, _):
        zv[pl.ds(r * 16, 16)] = jnp.zeros((16,), jnp.float32)
        return 0

    lax.fori_loop(0, ROWS_PER_SUB // 16, fill_zero, 0)
    base = s * ROWS_PER_SUB
    pltpu.sync_copy(zv, dssp.at[pl.ds(base, ROWS_PER_SUB)])
    pltpu.sync_copy(zv, ddsp.at[pl.ds(base, ROWS_PER_SUB)])
    plsc.subcore_barrier()

    def grp_s(g, _):
        for k in range(4):
            pltpu.async_copy(ones_v, dssp.at[idx_s.at[g * 4 + k]], sem, add=True)
        for k in range(4):
            pltpu.make_async_copy(ones_v, dssp.at[idx_s.at[g * 4 + k]], sem).wait()
        return 0

    lax.fori_loop(0, NCH // 4, grp_s, 0)

    def grp_d(g, _):
        for k in range(4):
            pltpu.async_copy(ones_v, ddsp.at[idx_d.at[g * 4 + k]], sem, add=True)
        for k in range(4):
            pltpu.make_async_copy(ones_v, ddsp.at[idx_d.at[g * 4 + k]], sem).wait()
        return 0

    lax.fori_loop(0, NCH // 4, grp_d, 0)
    plsc.subcore_barrier()
    pltpu.sync_copy(dssp.at[pl.ds(base, ROWS_PER_SUB)],
                    os_hbm.at[c, pl.ds(base, ROWS_PER_SUB)])
    pltpu.sync_copy(ddsp.at[pl.ds(base, ROWS_PER_SUB)],
                    od_hbm.at[c, pl.ds(base, ROWS_PER_SUB)])


@functools.partial(
    pl.kernel,
    out_type=jax.ShapeDtypeStruct((2, NP, D), jnp.float32),
    mesh=_mesh,
    scratch_types=[
        pltpu.VMEM((CH,), jnp.int32),
        pltpu.VMEM((CH,), jnp.int32),
        pltpu.VMEM((CH,), jnp.int32),
        pltpu.VMEM((CH,), jnp.int32),
        pltpu.VMEM((CH, D), jnp.float32),
        pltpu.VMEM((CH, D), jnp.float32),
        pltpu.VMEM_SHARED((NP, D), jnp.float32),
        pltpu.SemaphoreType.DMA,
        pltpu.SemaphoreType.DMA,
    ],
)
def _sc_agg(xn_hbm, src_hbm, dst_hbm, out_hbm,
            is0, is1, id0, id1, rows0, rows1, aggsp, isem, gsem):
    c = lax.axis_index("c")
    s = lax.axis_index("s")
    wid = s * 2 + c
    isb = (is0, is1)
    idb = (id0, id1)
    rows = (rows0, rows1)

    def fire_idx(j, b):
        pltpu.async_copy(src_hbm.at[wid, j], isb[b], isem)
        pltpu.async_copy(dst_hbm.at[wid, j], idb[b], isem)

    def wait_idx(j, b):
        pltpu.make_async_copy(src_hbm.at[wid, j], isb[b], isem).wait()
        pltpu.make_async_copy(dst_hbm.at[wid, j], idb[b], isem).wait()

    fire_idx(0, 0)
    fire_idx(1, 1)

    def zrow(r, _):
        for k in range(D // 16):
            rows0[r, pl.ds(k * 16, 16)] = jnp.zeros((16,), jnp.float32)
        return 0

    lax.fori_loop(0, CH, zrow, 0)
    base = s * ROWS_PER_SUB

    def zcp(k, _):
        pltpu.sync_copy(rows0, aggsp.at[pl.ds(base + k * CH, CH)])
        return 0

    lax.fori_loop(0, ROWS_PER_SUB // CH, zcp, 0)
    plsc.subcore_barrier()

    wait_idx(0, 0)
    pltpu.async_copy(xn_hbm.at[is0], rows0, gsem)

    def grp(g, _):
        for b in range(2):
            j = g * 2 + b
            pltpu.make_async_copy(xn_hbm.at[isb[b]], rows[b], gsem).wait()
            wait_idx(j + 1, 1 - b)
            pltpu.async_copy(xn_hbm.at[isb[1 - b]], rows[1 - b], gsem)
            pltpu.sync_copy(rows[b], aggsp.at[idb[b]], add=True)
            fire_idx(j + 2, b)
        return 0

    lax.fori_loop(0, NCH // 2, grp, 0)
    pltpu.make_async_copy(xn_hbm.at[is0], rows0, gsem).wait()
    wait_idx(NCH + 1, 1)
    plsc.subcore_barrier()
    pltpu.sync_copy(aggsp.at[pl.ds(base, ROWS_PER_SUB)],
                    out_hbm.at[c, pl.ds(base, ROWS_PER_SUB)])


BN = 1024


def _prep_body(ds_ref, x_ref, o_ref):
    ds = ds_ref[...]
    norm = lax.rsqrt(jnp.maximum(ds[:, 0:1] + ds[:, 1:2], 1.0))
    o_ref[...] = x_ref[...] * norm


def _prep(dsT, xp):
    return pl.pallas_call(
        _prep_body,
        grid=(NP // BN,),
        in_specs=[
            pl.BlockSpec((BN, 2), lambda i: (i, 0)),
            pl.BlockSpec((BN, D), lambda i: (i, 0)),
        ],
        out_specs=pl.BlockSpec((BN, D), lambda i: (i, 0)),
        out_shape=jax.ShapeDtypeStruct((NP, D), jnp.float32),
    )(dsT, xp)


def _l1_body(agg_ref, dd_ref, ds_ref, w_ref, b_ref, o_ref):
    a = agg_ref[0] + agg_ref[1]
    dd = dd_ref[...]
    ds = ds_ref[...]
    nd = lax.rsqrt(jnp.maximum(dd[:, 0:1] + dd[:, 1:2], 1.0))
    ns = lax.rsqrt(jnp.maximum(ds[:, 0:1] + ds[:, 1:2], 1.0))
    h = jnp.dot(a * nd, w_ref[...], preferred_element_type=jnp.float32)
    o_ref[...] = jnp.maximum(h + b_ref[...], 0.0) * ns


def _l1(agg, ddT, dsT, W1, b1):
    return pl.pallas_call(
        _l1_body,
        grid=(NP // BN,),
        in_specs=[
            pl.BlockSpec((2, BN, D), lambda i: (0, i, 0)),
            pl.BlockSpec((BN, 2), lambda i: (i, 0)),
            pl.BlockSpec((BN, 2), lambda i: (i, 0)),
            pl.BlockSpec((D, H), lambda i: (0, 0)),
            pl.BlockSpec((1, H), lambda i: (0, 0)),
        ],
        out_specs=pl.BlockSpec((BN, H), lambda i: (i, 0)),
        out_shape=jax.ShapeDtypeStruct((NP, H), jnp.float32),
    )(agg, ddT, dsT, W1, b1)


def _l2_body(agg_ref, dd_ref, w_ref, b_ref, wl_ref, bl_ref, o_ref, acc_ref):
    i = pl.program_id(0)
    a = agg_ref[0] + agg_ref[1]
    dd = dd_ref[...]
    nd = lax.rsqrt(jnp.maximum(dd[:, 0:1] + dd[:, 1:2], 1.0))
    y = jnp.dot(a * nd, w_ref[...], preferred_element_type=jnp.float32)
    y = y + b_ref[...]
    rows = i * BN + lax.broadcasted_iota(jnp.int32, (BN, 1), 0)
    y = jnp.where(rows < N, y, -jnp.inf)
    bm = jnp.max(y, axis=0, keepdims=True)

    @pl.when(i == 0)
    def _():
        acc_ref[...] = bm

    @pl.when(i > 0)
    def _():
        acc_ref[...] = jnp.maximum(acc_ref[...], bm)

    @pl.when(i == pl.num_programs(0) - 1)
    def _():
        pooled = jnp.maximum(acc_ref[...], 0.0)
        o_ref[...] = (
            jnp.dot(pooled, wl_ref[...], preferred_element_type=jnp.float32)
            + bl_ref[...]
        )


def _l2(agg, ddT, W2, b2, wlp, blp):
    return pl.pallas_call(
        _l2_body,
        grid=(NP // BN,),
        in_specs=[
            pl.BlockSpec((2, BN, D), lambda i: (0, i, 0)),
            pl.BlockSpec((BN, 2), lambda i: (i, 0)),
            pl.BlockSpec((D, H), lambda i: (0, 0)),
            pl.BlockSpec((1, H), lambda i: (0, 0)),
            pl.BlockSpec((H, 128), lambda i: (0, 0)),
            pl.BlockSpec((1, 128), lambda i: (0, 0)),
        ],
        out_specs=pl.BlockSpec((1, 128), lambda i: (0, 0)),
        out_shape=jax.ShapeDtypeStruct((1, 128), jnp.float32),
        scratch_shapes=[pltpu.VMEM((1, H), jnp.float32)],
    )(agg, ddT, W2, b2, wlp, blp)


def kernel(x, edge_index, W1, b1, W2, b2, Wl, bl):
    f32 = jnp.float32
    xp = jnp.zeros((NP, D), f32).at[:N].set(x)

    src = edge_index[0]
    dst = edge_index[1]
    npad = EPAD - E
    ar = jnp.arange(npad, dtype=jnp.int32)
    srcp = jnp.concatenate([src, N + 32 + (ar % 32)])
    dstp = jnp.concatenate([dst, N + (ar % 32)])
    dum = jnp.broadcast_to(
        jnp.arange(CH, dtype=jnp.int32)[None, None, :], (NW, 2, CH))
    src3 = jnp.concatenate([srcp.reshape(NW, NCH, CH), dum], axis=1)
    dst3 = jnp.concatenate([dstp.reshape(NW, NCH, CH), dum], axis=1)

    degs, degd = _sc_deg(src3, dst3)
    dsT = jnp.transpose(degs)
    ddT = jnp.transpose(degd)

    xn = _prep(dsT, xp)
    agg1 = _sc_agg(xn, src3, dst3)
    h1n = _l1(agg1, ddT, dsT, W1, b1.reshape(1, H))
    agg2 = _sc_agg(h1n, src3, dst3)

    wlp = jnp.zeros((H, 128), f32).at[:, :C].set(Wl)
    blp = jnp.zeros((1, 128), f32).at[:, :C].set(bl)
    res = _l2(agg2, ddT, W2, b2.reshape(1, H), wlp, blp)
    return res[:, :C]
